# baseline (speedup 1.0000x reference)
import jax
import jax.numpy as jnp
from jax import lax
from jax.experimental import pallas as pl
from jax.experimental.pallas import tpu as pltpu
from jax.experimental.pallas import tpu_sc as plsc

NC = 2
NS = 16
NW = NC * NS


def _sc_pool(text2, emb, B, L, D):
    SPW = B // NW
    C0 = 128
    C1 = L - C0

    def body(text_hbm, emb_hbm, pooled_hbm, idx_v, buf_a, buf_b, pooled_v,
             sem_a, sem_b):
        wid = lax.axis_index("s") * NC + lax.axis_index("c")
        base = wid * SPW

        pltpu.sync_copy(text_hbm.at[wid], idx_v)

        def gather_descs(s, buf, sem):
            off = pl.multiple_of(s * L, 8)
            d0 = pltpu.make_async_copy(
                emb_hbm.at[idx_v.at[pl.ds(off, C0)]],
                buf.at[pl.ds(0, C0)], sem)
            d1 = pltpu.make_async_copy(
                emb_hbm.at[idx_v.at[pl.ds(off + C0, C1)]],
                buf.at[pl.ds(C0, C1)], sem)
            return d0, d1

        def start(s, buf, sem):
            d0, d1 = gather_descs(s, buf, sem)
            d0.start()
            d1.start()

        def wait(s, buf, sem):
            d0, d1 = gather_descs(s, buf, sem)
            d0.wait()
            d1.wait()

        def reduce_store(s, buf):
            zero = jnp.zeros((16,), jnp.float32)

            def rbody(r, accs):
                out = list(accs)
                for u in range(8):
                    row = r * 8 + u
                    bank = (u % 2) * 4
                    for c in range(4):
                        out[bank + c] = out[bank + c] + buf[row,
                                                           pl.ds(c * 16, 16)]
                return tuple(out)

            accs = lax.fori_loop(0, L // 8, rbody, (zero,) * 8)
            scale = jnp.float32(1.0 / L)
            for c in range(4):
                pooled_v[s, pl.ds(c * 16, 16)] = (accs[c] + accs[4 + c]) * scale

        start(0, buf_a, sem_a)
        start(1, buf_b, sem_b)

        def pair(i, carry):
            s0 = 2 * i
            wait(s0, buf_a, sem_a)
            reduce_store(s0, buf_a)

            @pl.when(i < SPW // 2 - 1)
            def _():
                start(s0 + 2, buf_a, sem_a)

            wait(s0 + 1, buf_b, sem_b)
            reduce_store(s0 + 1, buf_b)

            @pl.when(i < SPW // 2 - 1)
            def _():
                start(s0 + 3, buf_b, sem_b)

            return carry

        lax.fori_loop(0, SPW // 2, pair, 0)

        pltpu.sync_copy(pooled_v, pooled_hbm.at[pl.ds(base, SPW)])

    return pl.kernel(
        body,
        out_type=jax.ShapeDtypeStruct((B, D), jnp.float32),
        mesh=plsc.VectorSubcoreMesh(core_axis_name="c", subcore_axis_name="s",
                                    num_cores=NC, num_subcores=NS),
        scratch_types=[
            pltpu.VMEM((SPW * L,), jnp.int32),
            pltpu.VMEM((L, D), jnp.float32),
            pltpu.VMEM((L, D), jnp.float32),
            pltpu.VMEM((SPW, D), jnp.float32),
            pltpu.SemaphoreType.DMA,
            pltpu.SemaphoreType.DMA,
        ],
        compiler_params=pltpu.CompilerParams(use_tc_tiling_on_sc=False),
    )(text2, emb)


def _mlp_body(x_ref, w1t_ref, b1_ref, w2t_ref, b2_ref, o_ref):
    h = jnp.dot(x_ref[...], w1t_ref[...],
                preferred_element_type=jnp.float32) + b1_ref[...]
    h = jnp.maximum(h, 0.0)
    o_ref[...] = jnp.dot(h, w2t_ref[...],
                         preferred_element_type=jnp.float32) + b2_ref[...]


def kernel(text, emb, W1, b1, W2, b2):
    B, L = text.shape
    V, D = emb.shape
    H = W1.shape[0]
    O = W2.shape[0]
    assert B % NW == 0 and (B // NW) % 2 == 0 and L % 8 == 0 and D == 64

    text2 = text.astype(jnp.int32).reshape(NW, (B // NW) * L)
    pooled = _sc_pool(text2, emb, B, L, D)

    OP = 128
    w2t_pad = jnp.zeros((H, OP), jnp.float32).at[:, :O].set(W2.T)
    b2_pad = jnp.zeros((1, OP), jnp.float32).at[:, :O].set(b2)

    out = pl.pallas_call(
        _mlp_body,
        out_shape=jax.ShapeDtypeStruct((B, OP), jnp.float32),
    )(pooled, W1.T, b1.reshape(1, H), w2t_pad, b2_pad)
    return out[:, :O]
